# baseline (speedup 1.0000x reference)
import functools
import jax
import jax.numpy as jnp
from jax import lax
from jax.experimental import pallas as pl
from jax.experimental.pallas import tpu as pltpu
from jax.experimental.pallas import tpu_sc as plsc

P = 4096
NPROP = 4000
G = 128
NG = 100
NFG = 128
NBG = 384
NS = 512
IOU_T = 0.5
NUM_CLASSES = 80


def _to_grid(x):
    return jnp.concatenate([x[:, r * 128:(r + 1) * 128] for r in range(32)],
                           axis=0)


def _dot3(a, b):
    f32 = jnp.float32
    hi = a.astype(jnp.bfloat16)
    r = a - hi.astype(f32)
    mid = r.astype(jnp.bfloat16)
    lo = (r - mid.astype(f32)).astype(jnp.bfloat16)
    bb = b.astype(jnp.bfloat16)
    dims = (((1,), (0,)), ((), ()))
    out = jax.lax.dot_general(hi, bb, dims, preferred_element_type=f32)
    out = out + jax.lax.dot_general(mid, bb, dims, preferred_element_type=f32)
    out = out + jax.lax.dot_general(lo, bb, dims, preferred_element_type=f32)
    return out


def _sortable_i32(s):
    bits = jax.lax.bitcast_convert_type(s, jnp.int32)
    return jnp.where(bits >= 0, bits, bits ^ jnp.int32(0x7FFFFFFF))


def _stage_a(prop_ref, gtc_ref, gtr_ref, keys_ref, feats_ref):
    f32 = jnp.float32
    pr = prop_ref[0]
    px1 = pr[0:1, :]
    py1 = pr[1:2, :]
    px2 = pr[2:3, :]
    py2 = pr[3:4, :]
    area2 = (px2 - px1) * (py2 - py1)

    def gt_chunk(c, carry):
        mv, mi = carry
        base = c * 32
        gch = gtc_ref[0, pl.ds(base, 32), :]
        gx1 = gch[:, 0:1]
        gy1 = gch[:, 1:2]
        gx2 = gch[:, 2:3]
        gy2 = gch[:, 3:4]
        area1 = (gx2 - gx1) * (gy2 - gy1)
        w = jnp.maximum(jnp.minimum(gx2, px2) - jnp.maximum(gx1, px1), 0.0)
        h = jnp.maximum(jnp.minimum(gy2, py2) - jnp.maximum(gy1, py1), 0.0)
        inter = w * h
        union = area1 + area2 - inter
        iou = inter / jnp.maximum(union, 1e-8)
        cmax = jnp.max(iou, axis=0, keepdims=True)
        rowi = jax.lax.broadcasted_iota(jnp.int32, (32, P), 0) + base
        cidx = jnp.min(jnp.where(iou == cmax, rowi, G), axis=0, keepdims=True)
        better = cmax > mv
        mi = jnp.where(better, cidx, mi)
        mv = jnp.maximum(mv, cmax)
        return mv, mi

    mv0 = jnp.full((1, P), -1.0, f32)
    mi0 = jnp.zeros((1, P), jnp.int32)
    mv, mi = jax.lax.fori_loop(0, 4, gt_chunk, (mv0, mi0))

    col = jax.lax.broadcasted_iota(jnp.int32, (1, P), 1)
    valid = col < NPROP
    fg = mv >= IOU_T
    s_fg = jnp.where(valid, jnp.where(fg, mv, -1.0), -2.0)
    s_bg = jnp.where(valid, jnp.where(fg, -1e9, -mv), -2e9)

    bias = jnp.int32(-2**31)
    keys_ref[0, 0:1, :] = ~_sortable_i32(s_fg) ^ bias
    keys_ref[0, 1:2, :] = ~_sortable_i32(s_bg) ^ bias

    gtr = gtr_ref[0]
    g_iota = jax.lax.broadcasted_iota(jnp.int32, (G, P), 0)
    oh_g = (g_iota == mi).astype(f32)
    pg = _dot3(gtr, oh_g)
    tx1 = pg[0:1, :]
    ty1 = pg[1:2, :]
    tx2 = pg[2:3, :]
    ty2 = pg[3:4, :]
    tcls = pg[4:5, :]

    sw = px2 - px1
    sh = py2 - py1
    scx = px1 + 0.5 * sw
    scy = py1 + 0.5 * sh
    tw = tx2 - tx1
    th = ty2 - ty1
    tcx = tx1 + 0.5 * tw
    tcy = ty1 + 0.5 * th
    dx = 10.0 * (tcx - scx) / jnp.maximum(sw, 1e-6)
    dy = 10.0 * (tcy - scy) / jnp.maximum(sh, 1e-6)
    dw = 5.0 * jnp.log(jnp.maximum(tw, 1e-6) / jnp.maximum(sw, 1e-6))
    dh = 5.0 * jnp.log(jnp.maximum(th, 1e-6) / jnp.maximum(sh, 1e-6))
    cls = jnp.where(fg, tcls, float(NUM_CLASSES))

    zero = jnp.zeros((1, P), f32)
    feats = jnp.concatenate(
        [dx, dy, dw, dh, cls, px1, py1, px2, py2,
         zero, zero, zero, zero, zero, zero, zero], axis=0)
    feats_ref[0] = jnp.transpose(feats, (1, 0))


def _sc_sort_body(keys_hbm, feats_hbm, out_hbm,
                  ka, kb, va, vb, off, idx_fg, idx_bg, rows_fg, rows_bg, sem):
    wid = lax.axis_index("s") * 2 + lax.axis_index("c")
    img = wid // 2
    typ = wid % 2
    lane = lax.iota(jnp.int32, 16)
    seg_base = lane * 256
    ones = jnp.ones((16,), jnp.int32)
    zeros = jnp.zeros((16,), jnp.int32)

    pltpu.sync_copy(keys_hbm.at[wid], ka)

    def init_body(t, _):
        va[pl.ds(t * 16, 16)] = lane + t * 16
        return 0
    lax.fori_loop(0, 256, init_body, 0)

    for p in range(4):
        src_k, src_v, dst_k, dst_v = (
            (ka, va, kb, vb) if p % 2 == 0 else (kb, vb, ka, va))
        shift = 8 * p

        def zbody(t, _):
            off[pl.ds(t * 16, 16)] = zeros
            return 0
        lax.fori_loop(0, 256, zbody, 0)

        def hbody(t, _):
            k = plsc.load_gather(src_k, [seg_base + t])
            d = (jnp.right_shift(k, shift) & 255)
            plsc.addupdate_scatter(off, [d * 16 + lane], ones)
            return 0
        lax.fori_loop(0, 256, hbody, 0)

        def obody(d, carry):
            row = off[pl.ds(d * 16, 16)]
            incl = plsc.cumsum(row)
            off[pl.ds(d * 16, 16)] = incl - row + carry
            return carry + jnp.sum(row)
        lax.fori_loop(0, 256, obody, jnp.int32(0))

        def pbody(t, _):
            idxv = seg_base + t
            k = plsc.load_gather(src_k, [idxv])
            v = plsc.load_gather(src_v, [idxv])
            slot = (jnp.right_shift(k, shift) & 255) * 16 + lane
            dst = plsc.load_gather(off, [slot])
            plsc.store_scatter(dst_k, [dst], k)
            plsc.store_scatter(dst_v, [dst], v)
            plsc.addupdate_scatter(off, [slot], ones)
            return 0
        lax.fori_loop(0, 256, pbody, 0)

    base = img * P

    @pl.when(typ == 0)
    def _():
        def fgi_body(j, _):
            idx_fg[pl.ds(j * 16, 16)] = va[pl.ds(j * 16, 16)] + base
            return 0
        lax.fori_loop(0, NFG // 16, fgi_body, 0)
        pltpu.async_copy(feats_hbm.at[idx_fg], rows_fg, sem).wait()
        pltpu.sync_copy(rows_fg, out_hbm.at[img, pl.ds(0, NFG)])

    @pl.when(typ == 1)
    def _():
        def bgi_body(j, _):
            idx_bg[pl.ds(j * 16, 16)] = va[pl.ds(j * 16, 16)] + base
            return 0
        lax.fori_loop(0, NBG // 16, bgi_body, 0)
        pltpu.async_copy(feats_hbm.at[idx_bg], rows_bg, sem).wait()
        pltpu.sync_copy(rows_bg, out_hbm.at[img, pl.ds(NFG, NBG)])


def _sc_topk_gather(keys2, featsT, n):
    mesh = plsc.VectorSubcoreMesh(core_axis_name="c", subcore_axis_name="s")
    fn = functools.partial(
        pl.kernel, mesh=mesh,
        compiler_params=pltpu.CompilerParams(needs_layout_passes=False,
                                             use_tc_tiling_on_sc=False),
        out_type=jax.ShapeDtypeStruct((n, NS, 16), jnp.float32),
        scratch_types=[
            pltpu.VMEM((P,), jnp.int32),
            pltpu.VMEM((P,), jnp.int32),
            pltpu.VMEM((P,), jnp.int32),
            pltpu.VMEM((P,), jnp.int32),
            pltpu.VMEM((P,), jnp.int32),
            pltpu.VMEM((NFG,), jnp.int32),
            pltpu.VMEM((NBG,), jnp.int32),
            pltpu.VMEM((NFG, 16), jnp.float32),
            pltpu.VMEM((NBG, 16), jnp.float32),
            pltpu.SemaphoreType.DMA,
        ],
    )(_sc_sort_body)
    return fn(keys2, featsT)


def kernel(proposal_boxes, gt_boxes, gt_classes):
    n = proposal_boxes.shape[0]
    pb = proposal_boxes.astype(jnp.float32)
    gb = gt_boxes.astype(jnp.float32)
    gc = gt_classes.astype(jnp.float32)

    propT = jnp.pad(jnp.transpose(pb, (0, 2, 1)),
                    ((0, 0), (0, 4), (0, P - NPROP)))
    gt_cols = jnp.concatenate([gb, gc[..., None]], axis=-1)
    gt_cols = jnp.pad(gt_cols, ((0, 0), (0, G - NG), (0, 3)))
    gt_rows = jnp.transpose(gt_cols, (0, 2, 1))

    keys, feats = pl.pallas_call(
        _stage_a,
        grid=(n,),
        in_specs=[
            pl.BlockSpec((1, 8, P), lambda i: (i, 0, 0)),
            pl.BlockSpec((1, G, 8), lambda i: (i, 0, 0)),
            pl.BlockSpec((1, 8, G), lambda i: (i, 0, 0)),
        ],
        out_specs=[
            pl.BlockSpec((1, 2, P), lambda i: (i, 0, 0)),
            pl.BlockSpec((1, P, 16), lambda i: (i, 0, 0)),
        ],
        out_shape=[
            jax.ShapeDtypeStruct((n, 2, P), jnp.int32),
            jax.ShapeDtypeStruct((n, P, 16), jnp.float32),
        ],
    )(propT, gt_cols, gt_rows)

    out = _sc_topk_gather(keys.reshape(2 * n, P),
                          feats.reshape(n * P, 16), n)

    deltas = out[:, :, 0:4]
    classes = jnp.round(out[:, :, 4]).astype(jnp.int32)
    boxes = out[:, :, 5:9]
    return deltas, classes, boxes


# speedup vs baseline: 1.1989x; 1.1989x over previous
import functools
import jax
import jax.numpy as jnp
from jax import lax
from jax.experimental import pallas as pl
from jax.experimental.pallas import tpu as pltpu
from jax.experimental.pallas import tpu_sc as plsc

P = 4096
NPROP = 4000
G = 104
NG = 100
NFG = 128
NBG = 384
NS = 512
IOU_T = 0.5
NUM_CLASSES = 80


def _to_grid(x):
    return jnp.concatenate([x[:, r * 128:(r + 1) * 128] for r in range(32)],
                           axis=0)


def _dot3(a, b):
    f32 = jnp.float32
    hi = a.astype(jnp.bfloat16)
    r = a - hi.astype(f32)
    mid = r.astype(jnp.bfloat16)
    lo = (r - mid.astype(f32)).astype(jnp.bfloat16)
    bb = b.astype(jnp.bfloat16)
    dims = (((1,), (0,)), ((), ()))
    out = jax.lax.dot_general(hi, bb, dims, preferred_element_type=f32)
    out = out + jax.lax.dot_general(mid, bb, dims, preferred_element_type=f32)
    out = out + jax.lax.dot_general(lo, bb, dims, preferred_element_type=f32)
    return out


def _sortable_i32(s):
    bits = jax.lax.bitcast_convert_type(s, jnp.int32)
    return jnp.where(bits >= 0, bits, bits ^ jnp.int32(0x7FFFFFFF))


def _stage_a(prop_ref, gtc_ref, gtr_ref, keys_ref, feats_ref):
    f32 = jnp.float32
    pr = prop_ref[0]
    px1 = pr[0:1, :]
    py1 = pr[1:2, :]
    px2 = pr[2:3, :]
    py2 = pr[3:4, :]
    area2 = (px2 - px1) * (py2 - py1)

    def gt_chunk(base, rows, carry):
        mv, mi = carry
        gch = gtc_ref[0, pl.ds(base, rows), :]
        gx1 = gch[:, 0:1]
        gy1 = gch[:, 1:2]
        gx2 = gch[:, 2:3]
        gy2 = gch[:, 3:4]
        area1 = (gx2 - gx1) * (gy2 - gy1)
        w = jnp.maximum(jnp.minimum(gx2, px2) - jnp.maximum(gx1, px1), 0.0)
        h = jnp.maximum(jnp.minimum(gy2, py2) - jnp.maximum(gy1, py1), 0.0)
        inter = w * h
        union = area1 + area2 - inter
        iou = inter / jnp.maximum(union, 1e-8)
        cmax = jnp.max(iou, axis=0, keepdims=True)
        rowi = jax.lax.broadcasted_iota(jnp.int32, (rows, P), 0) + base
        cidx = jnp.min(jnp.where(iou == cmax, rowi, G), axis=0, keepdims=True)
        better = cmax > mv
        mi = jnp.where(better, cidx, mi)
        mv = jnp.maximum(mv, cmax)
        return mv, mi

    mv0 = jnp.full((1, P), -1.0, f32)
    mi0 = jnp.zeros((1, P), jnp.int32)
    carry = gt_chunk(0, 56, (mv0, mi0))
    mv, mi = gt_chunk(56, 48, carry)

    col = jax.lax.broadcasted_iota(jnp.int32, (1, P), 1)
    valid = col < NPROP
    fg = mv >= IOU_T
    s_fg = jnp.where(valid, jnp.where(fg, mv, -1.0), -2.0)
    s_bg = jnp.where(valid, jnp.where(fg, -1e9, -mv), -2e9)

    bias = jnp.int32(-2**31)
    keys_ref[0, 0:1, :] = ~_sortable_i32(s_fg) ^ bias
    keys_ref[0, 1:2, :] = ~_sortable_i32(s_bg) ^ bias

    gtr = gtr_ref[0]
    g_iota = jax.lax.broadcasted_iota(jnp.int32, (G, P), 0)
    oh_g = (g_iota == mi).astype(f32)
    pg = _dot3(gtr, oh_g)
    tx1 = pg[0:1, :]
    ty1 = pg[1:2, :]
    tx2 = pg[2:3, :]
    ty2 = pg[3:4, :]
    tcls = pg[4:5, :]

    sw = px2 - px1
    sh = py2 - py1
    scx = px1 + 0.5 * sw
    scy = py1 + 0.5 * sh
    tw = tx2 - tx1
    th = ty2 - ty1
    tcx = tx1 + 0.5 * tw
    tcy = ty1 + 0.5 * th
    dx = 10.0 * (tcx - scx) / jnp.maximum(sw, 1e-6)
    dy = 10.0 * (tcy - scy) / jnp.maximum(sh, 1e-6)
    dw = 5.0 * jnp.log(jnp.maximum(tw, 1e-6) / jnp.maximum(sw, 1e-6))
    dh = 5.0 * jnp.log(jnp.maximum(th, 1e-6) / jnp.maximum(sh, 1e-6))
    cls = jnp.where(fg, tcls, float(NUM_CLASSES))

    zero = jnp.zeros((1, P), f32)
    feats_ref[0] = jnp.concatenate(
        [dx, dy, dw, dh, cls, px1, py1, px2, py2,
         zero, zero, zero, zero, zero, zero, zero], axis=0)


def _sc_sort_body(keys_hbm, out_hbm, ka, kb, va, vb, off):
    wid = lax.axis_index("s") * 2 + lax.axis_index("c")
    lane = lax.iota(jnp.int32, 16)
    seg_base = lane * 256
    ones = jnp.ones((16,), jnp.int32)
    zeros = jnp.zeros((16,), jnp.int32)

    pltpu.sync_copy(keys_hbm.at[wid], ka)

    def init_body(t, _):
        for u in range(8):
            va[pl.ds((t * 8 + u) * 16, 16)] = lane + (t * 8 + u) * 16
        return 0
    lax.fori_loop(0, 32, init_body, 0)

    for p in range(4):
        src_k, src_v, dst_k, dst_v = (
            (ka, va, kb, vb) if p % 2 == 0 else (kb, vb, ka, va))
        shift = 8 * p

        def zbody(t, _):
            for u in range(8):
                off[pl.ds((t * 8 + u) * 16, 16)] = zeros
            return 0
        lax.fori_loop(0, 32, zbody, 0)

        def hbody(t, _):
            for u in range(4):
                k = plsc.load_gather(src_k, [seg_base + (t * 4 + u)])
                d = (jnp.right_shift(k, shift) & 255)
                plsc.addupdate_scatter(off, [d * 16 + lane], ones)
            return 0
        lax.fori_loop(0, 64, hbody, 0)

        def obody(d, carry):
            for u in range(4):
                sl = pl.ds((d * 4 + u) * 16, 16)
                row = off[sl]
                incl = plsc.cumsum(row)
                off[sl] = incl - row + carry
                carry = carry + incl[15]
            return carry
        lax.fori_loop(0, 64, obody, jnp.int32(0))

        def pbody(t, _):
            for u in range(2):
                idxv = seg_base + (t * 2 + u)
                k = plsc.load_gather(src_k, [idxv])
                v = plsc.load_gather(src_v, [idxv])
                slot = (jnp.right_shift(k, shift) & 255) * 16 + lane
                dst = plsc.load_gather(off, [slot])
                plsc.store_scatter(dst_k, [dst], k)
                plsc.store_scatter(dst_v, [dst], v)
                plsc.addupdate_scatter(off, [slot], ones)
            return 0
        lax.fori_loop(0, 128, pbody, 0)

    pltpu.sync_copy(va.at[pl.ds(0, NS)], out_hbm.at[wid])


def _sc_topk(keys2):
    mesh = plsc.VectorSubcoreMesh(core_axis_name="c", subcore_axis_name="s")
    fn = functools.partial(
        pl.kernel, mesh=mesh,
        compiler_params=pltpu.CompilerParams(needs_layout_passes=False),
        out_type=jax.ShapeDtypeStruct((32, NS), jnp.int32),
        scratch_types=[
            pltpu.VMEM((P,), jnp.int32),
            pltpu.VMEM((P,), jnp.int32),
            pltpu.VMEM((P,), jnp.int32),
            pltpu.VMEM((P,), jnp.int32),
            pltpu.VMEM((P,), jnp.int32),
        ],
    )(_sc_sort_body)
    return fn(keys2)


def _stage_c(feats_ref, sidx_ref, out_ref):
    f32 = jnp.float32
    feats = feats_ref[0]
    sidx = sidx_ref[0]
    p_iota = jax.lax.broadcasted_iota(jnp.int32, (P, 128), 0)
    for c in range(4):
        chunk = sidx[:, c * 128:(c + 1) * 128]
        oh = (p_iota == chunk).astype(f32)
        sm = _dot3(feats, oh)
        out_ref[0, :, c * 128:(c + 1) * 128] = sm


def kernel(proposal_boxes, gt_boxes, gt_classes):
    n = proposal_boxes.shape[0]
    pb = proposal_boxes.astype(jnp.float32)
    gb = gt_boxes.astype(jnp.float32)
    gc = gt_classes.astype(jnp.float32)

    propT = jnp.pad(jnp.transpose(pb, (0, 2, 1)),
                    ((0, 0), (0, 4), (0, P - NPROP)))
    gt_cols = jnp.concatenate([gb, gc[..., None]], axis=-1)
    gt_cols = jnp.pad(gt_cols, ((0, 0), (0, G - NG), (0, 3)))
    gt_rows = jnp.transpose(gt_cols, (0, 2, 1))

    keys, feats = pl.pallas_call(
        _stage_a,
        grid=(n,),
        in_specs=[
            pl.BlockSpec((1, 8, P), lambda i: (i, 0, 0)),
            pl.BlockSpec((1, G, 8), lambda i: (i, 0, 0)),
            pl.BlockSpec((1, 8, G), lambda i: (i, 0, 0)),
        ],
        out_specs=[
            pl.BlockSpec((1, 2, P), lambda i: (i, 0, 0)),
            pl.BlockSpec((1, 16, P), lambda i: (i, 0, 0)),
        ],
        out_shape=[
            jax.ShapeDtypeStruct((n, 2, P), jnp.int32),
            jax.ShapeDtypeStruct((n, 16, P), jnp.float32),
        ],
    )(propT, gt_cols, gt_rows)

    sorted512 = _sc_topk(keys.reshape(2 * n, P))
    s3 = sorted512.reshape(n, 2, NS)
    sidx = jnp.concatenate([s3[:, 0, 0:NFG], s3[:, 1, 0:NBG]], axis=1)
    sidx = sidx[:, None, :]

    out = pl.pallas_call(
        _stage_c,
        grid=(n,),
        in_specs=[
            pl.BlockSpec((1, 16, P), lambda i: (i, 0, 0)),
            pl.BlockSpec((1, 1, NS), lambda i: (i, 0, 0)),
        ],
        out_specs=pl.BlockSpec((1, 16, NS), lambda i: (i, 0, 0)),
        out_shape=jax.ShapeDtypeStruct((n, 16, NS), jnp.float32),
    )(feats, sidx)

    deltas = jnp.transpose(out[:, 0:4, :], (0, 2, 1))
    classes = jnp.round(out[:, 4, :]).astype(jnp.int32)
    boxes = jnp.transpose(out[:, 5:9, :], (0, 2, 1))
    return deltas, classes, boxes
